# baseline (speedup 1.0000x reference)
import functools

import jax
import jax.numpy as jnp
from jax import lax
from jax.experimental import pallas as pl
from jax.experimental.pallas import tpu as pltpu
from jax.experimental.pallas import tpu_sc as plsc

_BINS = 1000000
_TOKEN_DIM = 64
_N_CTX = 8192
_B = 4

_NC = 2
_NS = 16
_NW = _NC * _NS
_TOK = _B * _N_CTX
_PER_W = _TOK // _NW
_LANES = 16


def _sc_body(tokens_hbm, x_emb_hbm, pos_emb_hbm, out_hbm, idx_v, rows_v, pos_v, sem):
    c = lax.axis_index("c")
    s = lax.axis_index("s")
    wid = s * _NC + c
    base = wid * _PER_W
    pos_base = (wid % (_N_CTX // _PER_W)) * _PER_W

    pltpu.sync_copy(tokens_hbm.at[pl.ds(base, _PER_W)], idx_v)
    gather = pltpu.async_copy(x_emb_hbm.at[idx_v], rows_v, sem)

    half = _PER_W // 2
    for h in range(2):
        pltpu.sync_copy(
            pos_emb_hbm.at[pl.ds(pos_base + h * half, half)], pos_v
        )
        if h == 0:
            gather.wait()

        def add_row(r, carry, off=h * half):
            for j in range(_TOKEN_DIM // _LANES):
                sl = pl.ds(j * _LANES, _LANES)
                rows_v[off + r, sl] = rows_v[off + r, sl] + pos_v[r, sl]
            return carry

        lax.fori_loop(0, half, add_row, 0)

    pltpu.sync_copy(rows_v, out_hbm.at[pl.ds(base, _PER_W)])


@jax.jit
def _conditioner(tokens_flat, x_emb, pos_emb):
    mesh = plsc.VectorSubcoreMesh(core_axis_name="c", subcore_axis_name="s")
    run = functools.partial(
        pl.kernel,
        mesh=mesh,
        out_type=jax.ShapeDtypeStruct((_TOK, _TOKEN_DIM), jnp.float32),
        scratch_types=[
            pltpu.VMEM((_PER_W,), jnp.int32),
            pltpu.VMEM((_PER_W, _TOKEN_DIM), jnp.float32),
            pltpu.VMEM((_PER_W // 2, _TOKEN_DIM), jnp.float32),
            pltpu.SemaphoreType.DMA,
        ],
        compiler_params=pltpu.CompilerParams(use_tc_tiling_on_sc=False),
    )(_sc_body)
    return run(tokens_flat, x_emb, pos_emb)


def kernel(tokens, x_emb, pos_emb):
    tokens_flat = tokens.reshape(-1).astype(jnp.int32)
    out = _conditioner(tokens_flat, x_emb, pos_emb)
    return out.reshape(_B, _N_CTX, _TOKEN_DIM)


# speedup vs baseline: 1.5131x; 1.5131x over previous
import functools

import jax
import jax.numpy as jnp
from jax import lax
from jax.experimental import pallas as pl
from jax.experimental.pallas import tpu as pltpu
from jax.experimental.pallas import tpu_sc as plsc

_BINS = 1000000
_TOKEN_DIM = 64
_N_CTX = 8192
_B = 4

_NC = 2
_NS = 16
_NW = _NC * _NS
_TOK = _B * _N_CTX
_PER_W = _TOK // _NW
_HALF = _PER_W // 2
_LANES = 16
_GRP = 16


def _sc_body(tokens_hbm, x_emb_hbm, pos_emb_hbm, out_hbm, idx_v, rows_v, pos_v, sem):
    c = lax.axis_index("c")
    s = lax.axis_index("s")
    wid = s * _NC + c
    base = wid * _PER_W
    pos_base = (wid % (_N_CTX // _PER_W)) * _PER_W

    pltpu.sync_copy(tokens_hbm.at[pl.ds(base, _PER_W)], idx_v)

    for h in range(2):
        hoff = h * _HALF

        def gather_grp(g, carry, hoff=hoff):
            t0 = g * _GRP
            toks = idx_v[pl.ds(hoff + t0, _GRP)]
            for k in range(_GRP):
                t = t0 + k
                pltpu.async_copy(
                    x_emb_hbm.at[pl.ds(toks[k], 1)], rows_v.at[pl.ds(t, 1)], sem
                )
            for k in range(_GRP):
                t = t0 + k
                pltpu.make_async_copy(
                    x_emb_hbm.at[pl.ds(0, 1)], rows_v.at[pl.ds(t, 1)], sem
                ).wait()
            return carry

        lax.fori_loop(0, _HALF // _GRP, gather_grp, 0)

        for q in range(2):
            pltpu.sync_copy(
                pos_emb_hbm.at[pl.ds(pos_base + hoff + q * (_HALF // 2), _HALF // 2)],
                pos_v,
            )

            def add_row(r, carry, off=q * (_HALF // 2)):
                for j in range(_TOKEN_DIM // _LANES):
                    sl = pl.ds(j * _LANES, _LANES)
                    rows_v[off + r, sl] = rows_v[off + r, sl] + pos_v[r, sl]
                return carry

            lax.fori_loop(0, _HALF // 2, add_row, 0)

        pltpu.sync_copy(rows_v, out_hbm.at[pl.ds(base + hoff, _HALF)])


@jax.jit
def _conditioner(tokens_flat, x_emb, pos_emb):
    mesh = plsc.VectorSubcoreMesh(core_axis_name="c", subcore_axis_name="s")
    run = functools.partial(
        pl.kernel,
        mesh=mesh,
        out_type=jax.ShapeDtypeStruct((_TOK, _TOKEN_DIM), jnp.float32),
        scratch_types=[
            pltpu.VMEM((_PER_W,), jnp.int32),
            pltpu.VMEM((_HALF, _TOKEN_DIM), jnp.float32),
            pltpu.VMEM((_HALF // 2, _TOKEN_DIM), jnp.float32),
            pltpu.SemaphoreType.DMA,
        ],
    )(_sc_body)
    return run(tokens_flat, x_emb, pos_emb)


def kernel(tokens, x_emb, pos_emb):
    tokens_flat = tokens.reshape(-1).astype(jnp.int32)
    out = _conditioner(tokens_flat, x_emb, pos_emb)
    return out.reshape(_B, _N_CTX, _TOKEN_DIM)


# speedup vs baseline: 1.7624x; 1.1647x over previous
import functools

import jax
import jax.numpy as jnp
from jax import lax
from jax.experimental import pallas as pl
from jax.experimental.pallas import tpu as pltpu
from jax.experimental.pallas import tpu_sc as plsc

_BINS = 1000000
_TOKEN_DIM = 64
_N_CTX = 8192
_B = 4

_NC = 2
_NS = 16
_NW = _NC * _NS
_TOK = _B * _N_CTX
_LANES = 16

_SLAB = 256
_NFULL = _BINS // _SLAB
_REM = _BINS - _NFULL * _SLAB
_REM_OWNER = _NFULL % _NW
_CHUNK = 2048
_RING = 8


def _iota16():
    return jax.lax.broadcasted_iota(jnp.int32, (_LANES,), 0)


def _gather_body(tokens_hbm, xt_hbm, tail_hbm, raw_hbm, surv_tok, surv_pos,
                 tokch, slab_a, slab_b, hit_m, hit_p, rowbuf, tail_v,
                 sem_a, sem_b, sem_row):
    c = lax.axis_index("c")
    s = lax.axis_index("s")
    wid = s * _NC + c

    @pl.when(wid == _REM_OWNER)
    def _load_tail():
        pltpu.sync_copy(tail_hbm, tail_v)

    def scan_chunk(ch, cur):
        pltpu.sync_copy(tokens_hbm.at[pl.ds(ch * _CHUNK, _CHUNK)], tokch)

        def scan_vreg(g, cur):
            toks = tokch[pl.ds(g * _LANES, _LANES)]
            slab = lax.shift_right_logical(toks, 8)
            mask = lax.bitwise_and(slab, _NW - 1) == wid
            cnt = plsc.all_reduce_population_count(mask)[0]

            @pl.when(cnt > 0)
            def _append():
                pos = _iota16() + (ch * _CHUNK + g * _LANES)
                plsc.store_compressed(
                    surv_tok.at[pl.ds(cur, _LANES)], toks, mask=mask)
                plsc.store_compressed(
                    surv_pos.at[pl.ds(cur, _LANES)], pos, mask=mask)

            return cur + cnt

        return lax.fori_loop(0, _CHUNK // _LANES, scan_vreg, cur)

    nsurv = lax.fori_loop(0, _TOK // _CHUNK, scan_chunk, 0)
    nsv = lax.shift_right_logical(nsurv + _LANES - 1, 4)

    def issue_slab(k, buf, sem):
        start = (wid + k * _NW) * _SLAB
        pltpu.async_copy(
            xt_hbm.at[:, pl.ds(pl.multiple_of(start, 128), _SLAB)], buf, sem)

    def wait_slab(buf, sem):
        pltpu.make_async_copy(
            xt_hbm.at[:, pl.ds(0, _SLAB)], buf, sem).wait()

    def process_slab(buf, si, carry):
        def do_hit(h, pending):
            mvec = hit_m[pl.ds(h, _LANES)]
            pvec = hit_p[pl.ds(h, _LANES)]
            m = mvec[0]
            p = pvec[0]

            @pl.when(pending == _RING)
            def _absorb():
                pltpu.make_async_copy(
                    rowbuf, raw_hbm.at[pl.ds(0, _RING)], sem_row).wait()

            slot = jnp.where(pending == _RING, 0, pending)
            col = jnp.full((_LANES,), m, dtype=jnp.int32)
            for j in range(_TOKEN_DIM // _LANES):
                vals = plsc.load_gather(buf, [_iota16() + j * _LANES, col])
                rowbuf[slot, pl.ds(j * _LANES, _LANES)] = vals
            pltpu.async_copy(
                rowbuf.at[pl.ds(slot, 1)], raw_hbm.at[pl.ds(p, 1)], sem_row)
            return jnp.where(pending == _RING, 1, pending + 1)

        def scan_surv(g, pending):
            toks = surv_tok[pl.ds(g * _LANES, _LANES)]
            poss = surv_pos[pl.ds(g * _LANES, _LANES)]
            valid = (_iota16() + g * _LANES) < nsurv
            mask = jnp.logical_and(lax.shift_right_logical(toks, 8) == si, valid)
            cnt = plsc.all_reduce_population_count(mask)[0]
            mloc = toks - si * _SLAB
            plsc.store_compressed(hit_m.at[pl.ds(0, _LANES)], mloc, mask=mask)
            plsc.store_compressed(hit_p.at[pl.ds(0, _LANES)], poss, mask=mask)
            return lax.fori_loop(0, cnt, do_hit, pending)

        return lax.fori_loop(0, nsv, scan_surv, carry)

    my_nfull = lax.shift_right_logical(_NFULL - 1 - wid, 5) + 1
    npairs = lax.shift_right_logical(my_nfull + 1, 1)
    def k_start(k):
        kk = jnp.where(k < my_nfull, k, 0)
        return (wid + kk * _NW) * _SLAB

    def k_si(k):
        return jnp.where(k < my_nfull, wid + k * _NW, -1)

    def issue_k(k, buf, sem):
        pltpu.async_copy(
            xt_hbm.at[:, pl.ds(pl.multiple_of(k_start(k), 128), _SLAB)],
            buf, sem)

    issue_k(0, slab_a, sem_a)

    def pair(i, pending):
        k0 = 2 * i
        issue_k(k0 + 1, slab_b, sem_b)
        wait_slab(slab_a, sem_a)
        pending = process_slab(slab_a, k_si(k0), pending)

        @pl.when(i + 1 < npairs)
        def _pre_a():
            issue_k(k0 + 2, slab_a, sem_a)

        wait_slab(slab_b, sem_b)
        return process_slab(slab_b, k_si(k0 + 1), pending)

    pending = lax.fori_loop(0, npairs, pair, 0)

    def do_tail_hit(h, pending):
        m = hit_m[pl.ds(h, _LANES)][0]
        p = hit_p[pl.ds(h, _LANES)][0]

        @pl.when(pending == _RING)
        def _absorb():
            pltpu.make_async_copy(
                rowbuf, raw_hbm.at[pl.ds(0, _RING)], sem_row).wait()

        slot = jnp.where(pending == _RING, 0, pending)
        for j in range(_TOKEN_DIM // _LANES):
            sl = pl.ds(j * _LANES, _LANES)
            rowbuf[slot, sl] = tail_v[m, sl]
        pltpu.async_copy(
            rowbuf.at[pl.ds(slot, 1)], raw_hbm.at[pl.ds(p, 1)], sem_row)
        return jnp.where(pending == _RING, 1, pending + 1)

    def tail_scan(g, pending):
        toks = surv_tok[pl.ds(g * _LANES, _LANES)]
        poss = surv_pos[pl.ds(g * _LANES, _LANES)]
        valid = (_iota16() + g * _LANES) < nsurv
        mask = jnp.logical_and(
            jnp.logical_and(lax.shift_right_logical(toks, 8) == _NFULL, valid),
            wid == _REM_OWNER)
        cnt = plsc.all_reduce_population_count(mask)[0]
        mloc = toks - _NFULL * _SLAB
        plsc.store_compressed(hit_m.at[pl.ds(0, _LANES)], mloc, mask=mask)
        plsc.store_compressed(hit_p.at[pl.ds(0, _LANES)], poss, mask=mask)
        return lax.fori_loop(0, cnt, do_tail_hit, pending)

    pending = lax.fori_loop(0, nsv, tail_scan, pending)

    def drain(r, carry):
        @pl.when(r < pending)
        def _w():
            pltpu.make_async_copy(
                rowbuf.at[pl.ds(0, 1)], raw_hbm.at[pl.ds(0, 1)], sem_row
            ).wait()
        return carry

    lax.fori_loop(0, _RING, drain, 0)


def _add_body(raw_hbm, pos_hbm, out_hbm, rows_v, pos_v):
    c = lax.axis_index("c")
    s = lax.axis_index("s")
    wid = s * _NC + c
    per_w = _TOK // _NW
    base = wid * per_w
    pos_base = (wid % (_N_CTX // per_w)) * per_w
    q = per_w // 4

    for h in range(4):
        pltpu.sync_copy(raw_hbm.at[pl.ds(base + h * q, q)], rows_v)
        pltpu.sync_copy(pos_hbm.at[pl.ds(pos_base + h * q, q)], pos_v)

        def add_row(r, carry):
            for j in range(_TOKEN_DIM // _LANES):
                sl = pl.ds(j * _LANES, _LANES)
                rows_v[r, sl] = rows_v[r, sl] + pos_v[r, sl]
            return carry

        lax.fori_loop(0, q, add_row, 0)
        pltpu.sync_copy(rows_v, out_hbm.at[pl.ds(base + h * q, q)])


@jax.jit
def _conditioner(tokens_flat, xt, tail, pos_emb):
    mesh = plsc.VectorSubcoreMesh(core_axis_name="c", subcore_axis_name="s")
    raw = functools.partial(
        pl.kernel,
        mesh=mesh,
        compiler_params=pltpu.CompilerParams(needs_layout_passes=False),
        out_type=jax.ShapeDtypeStruct((_TOK, _TOKEN_DIM), jnp.float32),
        scratch_types=[
            pltpu.VMEM((_TOK + _LANES,), jnp.int32),
            pltpu.VMEM((_TOK + _LANES,), jnp.int32),
            pltpu.VMEM((_CHUNK,), jnp.int32),
            pltpu.VMEM((_TOKEN_DIM, _SLAB), jnp.float32),
            pltpu.VMEM((_TOKEN_DIM, _SLAB), jnp.float32),
            pltpu.VMEM((2 * _LANES,), jnp.int32),
            pltpu.VMEM((2 * _LANES,), jnp.int32),
            pltpu.VMEM((_RING, _TOKEN_DIM), jnp.float32),
            pltpu.VMEM((_REM, _TOKEN_DIM), jnp.float32),
            pltpu.SemaphoreType.DMA,
            pltpu.SemaphoreType.DMA,
            pltpu.SemaphoreType.DMA,
        ],
    )(_gather_body)(tokens_flat, xt, tail)

    out = functools.partial(
        pl.kernel,
        mesh=mesh,
        compiler_params=pltpu.CompilerParams(needs_layout_passes=False),
        out_type=jax.ShapeDtypeStruct((_TOK, _TOKEN_DIM), jnp.float32),
        scratch_types=[
            pltpu.VMEM((_TOK // _NW // 4, _TOKEN_DIM), jnp.float32),
            pltpu.VMEM((_TOK // _NW // 4, _TOKEN_DIM), jnp.float32),
        ],
    )(_add_body)(raw, pos_emb)
    return out


def kernel(tokens, x_emb, pos_emb):
    tokens_flat = tokens.reshape(-1).astype(jnp.int32)
    tail = x_emb[_NFULL * _SLAB:, :]
    out = _conditioner(tokens_flat, x_emb.T, tail, pos_emb)
    return out.reshape(_B, _N_CTX, _TOKEN_DIM)


# speedup vs baseline: 1.9274x; 1.0936x over previous
import functools

import jax
import jax.numpy as jnp
from jax import lax
from jax.experimental import pallas as pl
from jax.experimental.pallas import tpu as pltpu
from jax.experimental.pallas import tpu_sc as plsc

_BINS = 1000000
_TOKEN_DIM = 64
_N_CTX = 8192
_B = 4

_NC = 2
_NS = 16
_NW = _NC * _NS
_TOK = _B * _N_CTX
_LANES = 16

_SLAB = 256
_NSLAB = (_BINS + _SLAB - 1) // _SLAB
_TAIL_BASE = (_NSLAB - 1) * _SLAB
_TAIL_OWNER = (_NSLAB - 1) % _NW
_KMAX = (_NSLAB - 1) // _NW + 1
_NSLOT = _KMAX * _SLAB
_DUMP = _NSLOT
_CHUNK = 2048
_RING = 8
_PCAP = 2048


def _iota16():
    return jax.lax.broadcasted_iota(jnp.int32, (_LANES,), 0)


def _gather_body(tokens_hbm, xt_hbm, tail_hbm, raw_hbm, pend_hbm,
                 slot, tokch, slab_a, slab_b, hit_m, hit_p, rowbuf, tail_v,
                 pend_src, pend_dst, rowpend_s,
                 sem_a, sem_b, sem_row, sem_p):
    c = lax.axis_index("c")
    s = lax.axis_index("s")
    wid = s * _NC + c

    @pl.when(wid == _TAIL_OWNER)
    def _load_tail():
        pltpu.sync_copy(tail_hbm, tail_v)

    def memset(i, carry):
        slot[pl.ds(i * _LANES, _LANES)] = jnp.full((_LANES,), -1, jnp.int32)
        return carry

    lax.fori_loop(0, (_NSLOT + _LANES) // _LANES, memset, 0)
    rowpend_s[0] = 0

    def lanes_of(toks):
        sl = lax.shift_right_logical(toks, 8)
        mine = lax.bitwise_and(sl, _NW - 1) == wid
        k = lax.shift_right_logical(sl, 5)
        l = lax.bitwise_or(lax.shift_left(k, 8),
                           lax.bitwise_and(toks, _SLAB - 1))
        return jnp.where(mine, l, _DUMP), mine

    def a1_chunk(ch, carry):
        pltpu.sync_copy(tokens_hbm.at[pl.ds(ch * _CHUNK, _CHUNK)], tokch)

        def a1_vreg(g, carry):
            toks = tokch[pl.ds(g * _LANES, _LANES)]
            l, _ = lanes_of(toks)
            pos = _iota16() + (ch * _CHUNK + g * _LANES)
            plsc.store_scatter(slot, [l], pos)
            return carry

        return lax.fori_loop(0, _CHUNK // _LANES, a1_vreg, carry)

    lax.fori_loop(0, _TOK // _CHUNK, a1_chunk, 0)

    def a2_chunk(ch, carry):
        pltpu.sync_copy(tokens_hbm.at[pl.ds(ch * _CHUNK, _CHUNK)], tokch)

        def a2_vreg(g, carry):
            pcur, nspill = carry
            toks = tokch[pl.ds(g * _LANES, _LANES)]
            l, mine = lanes_of(toks)
            pos = _iota16() + (ch * _CHUNK + g * _LANES)
            got = plsc.load_gather(slot, [l])
            lost = jnp.logical_and(mine, got != pos)
            lcnt = plsc.all_reduce_population_count(lost)[0]

            def append(carry):
                pcur, nspill = carry
                li = jnp.where(lost, 1, 0)
                for k in range(_LANES):
                    idxv = jnp.where(
                        jnp.logical_and(_iota16() == k, lost),
                        jnp.full((_LANES,), pcur, jnp.int32),
                        jnp.full((_LANES,), _PCAP, jnp.int32))
                    plsc.store_scatter(pend_src, [idxv], got)
                    plsc.store_scatter(pend_dst, [idxv], pos)
                    pcur = pcur + li[k]

                def spill(cc):
                    pc, ns = cc
                    pltpu.sync_copy(
                        pend_src.at[pl.ds(0, _PCAP)],
                        pend_hbm.at[0, pl.ds(ns * _PCAP, _PCAP)])
                    pltpu.sync_copy(
                        pend_dst.at[pl.ds(0, _PCAP)],
                        pend_hbm.at[1, pl.ds(ns * _PCAP, _PCAP)])
                    return 0, ns + 1

                return lax.cond(pcur >= _PCAP, spill, lambda cc: cc,
                                (pcur, nspill))

            return lax.cond(lcnt > 0, append, lambda cc: cc, (pcur, nspill))

        return lax.fori_loop(0, _CHUNK // _LANES, a2_vreg, carry)

    pcur, nspill = lax.fori_loop(0, _TOK // _CHUNK, a2_chunk, (0, 0))

    my_nfull = lax.shift_right_logical(_NSLAB - 2 - wid, 5) + 1
    npairs = lax.shift_right_logical(my_nfull + 1, 1)

    def k_start(k):
        kk = jnp.where(k < my_nfull, k, 0)
        return (wid + kk * _NW) * _SLAB

    def issue_k(k, buf, sem):
        pltpu.async_copy(
            xt_hbm.at[:, pl.ds(pl.multiple_of(k_start(k), 128), _SLAB)],
            buf, sem)

    def wait_slab(buf, sem):
        pltpu.make_async_copy(
            xt_hbm.at[:, pl.ds(0, _SLAB)], buf, sem).wait()

    def row_out(p):
        pd = rowpend_s[0]
        pltpu.async_copy(
            rowbuf.at[pl.ds(jnp.where(pd == _RING, 0, pd), 1)],
            raw_hbm.at[pl.ds(p, 1)], sem_row)
        rowpend_s[0] = jnp.where(pd == _RING, 1, pd + 1)

    def ring_prepare():
        pd = rowpend_s[0]

        @pl.when(pd == _RING)
        def _absorb():
            pltpu.make_async_copy(
                rowbuf, raw_hbm.at[pl.ds(0, _RING)], sem_row).wait()

        return jnp.where(pd == _RING, 0, pd)

    def process_region(k, gather_from_tail, buf):
        def scan_vreg(v, carry):
            base = lax.shift_left(k, 8) + v * _LANES
            sv = slot[pl.ds(base, _LANES)]
            hits = sv >= 0
            cnt = plsc.all_reduce_population_count(hits)[0]

            @pl.when(cnt > 0)
            def _hits():
                mvec = _iota16() + v * _LANES
                plsc.store_compressed(
                    hit_m.at[pl.ds(0, _LANES)], mvec, mask=hits)
                plsc.store_compressed(
                    hit_p.at[pl.ds(0, _LANES)], sv, mask=hits)

                def do_hit(h, carry):
                    m = hit_m[pl.ds(h, _LANES)][0]
                    p = hit_p[pl.ds(h, _LANES)][0]
                    slot_r = ring_prepare()
                    if gather_from_tail:
                        for j in range(_TOKEN_DIM // _LANES):
                            sl = pl.ds(j * _LANES, _LANES)
                            rowbuf[slot_r, sl] = tail_v[m, sl]
                    else:
                        col = jnp.full((_LANES,), m, dtype=jnp.int32)
                        for j in range(_TOKEN_DIM // _LANES):
                            vals = plsc.load_gather(
                                buf, [_iota16() + j * _LANES, col])
                            rowbuf[slot_r, pl.ds(j * _LANES, _LANES)] = vals
                    row_out(p)
                    return carry

                lax.fori_loop(0, cnt, do_hit, 0)

            return carry

        lax.fori_loop(0, _SLAB // _LANES, scan_vreg, 0)

    issue_k(0, slab_a, sem_a)

    def pairloop(i, carry):
        k0 = 2 * i
        issue_k(k0 + 1, slab_b, sem_b)
        wait_slab(slab_a, sem_a)

        @pl.when(k0 < my_nfull)
        def _pa():
            process_region(k0, False, slab_a)

        @pl.when(i + 1 < npairs)
        def _pre_a():
            issue_k(k0 + 2, slab_a, sem_a)

        wait_slab(slab_b, sem_b)

        @pl.when(k0 + 1 < my_nfull)
        def _pb():
            process_region(k0 + 1, False, slab_b)

        return carry

    lax.fori_loop(0, npairs, pairloop, 0)

    @pl.when(wid == _TAIL_OWNER)
    def _tail():
        process_region((_NSLAB - 1) // _NW, True, slab_a)

    def drain(r, carry):
        @pl.when(r < rowpend_s[0])
        def _w():
            pltpu.make_async_copy(
                rowbuf.at[pl.ds(0, 1)], raw_hbm.at[pl.ds(0, 1)], sem_row
            ).wait()
        return carry

    lax.fori_loop(0, _RING, drain, 0)

    def dup_copy(i, carry):
        src = pend_src[pl.ds(i, _LANES)][0]
        dst = pend_dst[pl.ds(i, _LANES)][0]
        pltpu.async_copy(
            raw_hbm.at[pl.ds(src, 1)], raw_hbm.at[pl.ds(dst, 1)], sem_p)
        pltpu.make_async_copy(
            raw_hbm.at[pl.ds(0, 1)], raw_hbm.at[pl.ds(0, 1)], sem_p).wait()
        return carry

    def dup_chunk(sp, carry):
        pltpu.sync_copy(pend_hbm.at[0, pl.ds(sp * _PCAP, _PCAP)],
                        pend_src.at[pl.ds(0, _PCAP)])
        pltpu.sync_copy(pend_hbm.at[1, pl.ds(sp * _PCAP, _PCAP)],
                        pend_dst.at[pl.ds(0, _PCAP)])
        lax.fori_loop(0, _PCAP, dup_copy, 0)
        return carry

    lax.fori_loop(0, nspill, dup_chunk, 0)
    lax.fori_loop(0, pcur, dup_copy, 0)


def _add_body(raw_hbm, pos_hbm, out_hbm, rows_v, pos_v):
    c = lax.axis_index("c")
    s = lax.axis_index("s")
    wid = s * _NC + c
    per_w = _TOK // _NW
    base = wid * per_w
    pos_base = (wid % (_N_CTX // per_w)) * per_w
    q = per_w // 4

    for h in range(4):
        pltpu.sync_copy(raw_hbm.at[pl.ds(base + h * q, q)], rows_v)
        pltpu.sync_copy(pos_hbm.at[pl.ds(pos_base + h * q, q)], pos_v)

        def add_row(r, carry):
            for j in range(_TOKEN_DIM // _LANES):
                sl = pl.ds(j * _LANES, _LANES)
                rows_v[r, sl] = rows_v[r, sl] + pos_v[r, sl]
            return carry

        lax.fori_loop(0, q, add_row, 0)
        pltpu.sync_copy(rows_v, out_hbm.at[pl.ds(base + h * q, q)])


@jax.jit
def _conditioner(tokens_flat, xt, tail, pos_emb):
    mesh = plsc.VectorSubcoreMesh(core_axis_name="c", subcore_axis_name="s")
    raw, _ = functools.partial(
        pl.kernel,
        mesh=mesh,
        compiler_params=pltpu.CompilerParams(needs_layout_passes=False),
        out_type=(
            jax.ShapeDtypeStruct((_TOK, _TOKEN_DIM), jnp.float32),
            jax.ShapeDtypeStruct((2, _TOK), jnp.int32),
        ),
        scratch_types=[
            pltpu.VMEM((_NSLOT + _LANES,), jnp.int32),
            pltpu.VMEM((_CHUNK,), jnp.int32),
            pltpu.VMEM((_TOKEN_DIM, _SLAB), jnp.float32),
            pltpu.VMEM((_TOKEN_DIM, _SLAB), jnp.float32),
            pltpu.VMEM((2 * _LANES,), jnp.int32),
            pltpu.VMEM((2 * _LANES,), jnp.int32),
            pltpu.VMEM((_RING, _TOKEN_DIM), jnp.float32),
            pltpu.VMEM((64, _TOKEN_DIM), jnp.float32),
            pltpu.VMEM((_PCAP + _LANES,), jnp.int32),
            pltpu.VMEM((_PCAP + _LANES,), jnp.int32),
            pltpu.SMEM((1,), jnp.int32),
            pltpu.SemaphoreType.DMA,
            pltpu.SemaphoreType.DMA,
            pltpu.SemaphoreType.DMA,
            pltpu.SemaphoreType.DMA,
        ],
    )(_gather_body)(tokens_flat, xt, tail)

    out = functools.partial(
        pl.kernel,
        mesh=mesh,
        compiler_params=pltpu.CompilerParams(needs_layout_passes=False),
        out_type=jax.ShapeDtypeStruct((_TOK, _TOKEN_DIM), jnp.float32),
        scratch_types=[
            pltpu.VMEM((_TOK // _NW // 4, _TOKEN_DIM), jnp.float32),
            pltpu.VMEM((_TOK // _NW // 4, _TOKEN_DIM), jnp.float32),
        ],
    )(_add_body)(raw, pos_emb)
    return out


def kernel(tokens, x_emb, pos_emb):
    tokens_flat = tokens.reshape(-1).astype(jnp.int32)
    tail = x_emb[_TAIL_BASE:, :]
    out = _conditioner(tokens_flat, x_emb.T, tail, pos_emb)
    return out.reshape(_B, _N_CTX, _TOKEN_DIM)


# speedup vs baseline: 2.1039x; 1.0916x over previous
import functools

import jax
import jax.numpy as jnp
from jax import lax
from jax.experimental import pallas as pl
from jax.experimental.pallas import tpu as pltpu
from jax.experimental.pallas import tpu_sc as plsc

_BINS = 1000000
_TOKEN_DIM = 64
_N_CTX = 8192
_B = 4

_NC = 2
_NS = 16
_NW = _NC * _NS
_TOK = _B * _N_CTX
_LANES = 16

_SLAB = 256
_NSLAB = (_BINS + _SLAB - 1) // _SLAB
_TAIL_BASE = (_NSLAB - 1) * _SLAB
_TAIL_OWNER = (_NSLAB - 1) % _NW
_KMAX = (_NSLAB - 1) // _NW + 1
_NSLOT = _KMAX * _SLAB
_DUMP = _NSLOT
_CHUNK = 2048
_RING = 8
_PCAP = 2048


def _iota16():
    return jax.lax.broadcasted_iota(jnp.int32, (_LANES,), 0)


def _gather_body(tokens_hbm, xt_hbm, tail_hbm, raw_hbm, pend_hbm,
                 slot, tokch, slab_a, slab_b, slab_c, slab_d,
                 hit_m, hit_p, rowbuf, tail_v,
                 pend_src, pend_dst, rowpend_s,
                 sem_a, sem_b, sem_c, sem_d, sem_row, sem_p):
    c = lax.axis_index("c")
    s = lax.axis_index("s")
    wid = s * _NC + c

    @pl.when(wid == _TAIL_OWNER)
    def _load_tail():
        pltpu.sync_copy(tail_hbm, tail_v)

    def memset(i, carry):
        slot[pl.ds(i * _LANES, _LANES)] = jnp.full((_LANES,), -1, jnp.int32)
        return carry

    lax.fori_loop(0, (_NSLOT + _LANES) // _LANES, memset, 0)
    rowpend_s[0] = 0

    def lanes_of(toks):
        sl = lax.shift_right_logical(toks, 8)
        mine = lax.bitwise_and(sl, _NW - 1) == wid
        k = lax.shift_right_logical(sl, 5)
        l = lax.bitwise_or(lax.shift_left(k, 8),
                           lax.bitwise_and(toks, _SLAB - 1))
        return jnp.where(mine, l, _DUMP), mine

    def a1_chunk(ch, carry):
        pltpu.sync_copy(tokens_hbm.at[pl.ds(ch * _CHUNK, _CHUNK)], tokch)

        def a1_vreg(g, carry):
            toks = tokch[pl.ds(g * _LANES, _LANES)]
            l, _ = lanes_of(toks)
            pos = _iota16() + (ch * _CHUNK + g * _LANES)
            plsc.store_scatter(slot, [l], pos)
            return carry

        return lax.fori_loop(0, _CHUNK // _LANES, a1_vreg, carry)

    lax.fori_loop(0, _TOK // _CHUNK, a1_chunk, 0)

    def a2_chunk(ch, carry):
        pltpu.sync_copy(tokens_hbm.at[pl.ds(ch * _CHUNK, _CHUNK)], tokch)

        def a2_vreg(g, carry):
            pcur, nspill = carry
            toks = tokch[pl.ds(g * _LANES, _LANES)]
            l, mine = lanes_of(toks)
            pos = _iota16() + (ch * _CHUNK + g * _LANES)
            got = plsc.load_gather(slot, [l])
            lost = jnp.logical_and(mine, got != pos)
            lcnt = plsc.all_reduce_population_count(lost)[0]

            def append(carry):
                pcur, nspill = carry
                li = jnp.where(lost, 1, 0)
                for k in range(_LANES):
                    idxv = jnp.where(
                        jnp.logical_and(_iota16() == k, lost),
                        jnp.full((_LANES,), pcur, jnp.int32),
                        jnp.full((_LANES,), _PCAP, jnp.int32))
                    plsc.store_scatter(pend_src, [idxv], got)
                    plsc.store_scatter(pend_dst, [idxv], pos)
                    pcur = pcur + li[k]

                def spill(cc):
                    pc, ns = cc
                    pltpu.sync_copy(
                        pend_src.at[pl.ds(0, _PCAP)],
                        pend_hbm.at[0, pl.ds(ns * _PCAP, _PCAP)])
                    pltpu.sync_copy(
                        pend_dst.at[pl.ds(0, _PCAP)],
                        pend_hbm.at[1, pl.ds(ns * _PCAP, _PCAP)])
                    return 0, ns + 1

                return lax.cond(pcur >= _PCAP, spill, lambda cc: cc,
                                (pcur, nspill))

            return lax.cond(lcnt > 0, append, lambda cc: cc, (pcur, nspill))

        return lax.fori_loop(0, _CHUNK // _LANES, a2_vreg, carry)

    pcur, nspill = lax.fori_loop(0, _TOK // _CHUNK, a2_chunk, (0, 0))

    my_nfull = lax.shift_right_logical(_NSLAB - 2 - wid, 5) + 1
    npairs = lax.shift_right_logical(my_nfull + 1, 1)

    def k_start(k):
        kk = jnp.where(k < my_nfull, k, 0)
        return (wid + kk * _NW) * _SLAB

    def issue_k(k, buf, sem):
        pltpu.async_copy(
            xt_hbm.at[:, pl.ds(pl.multiple_of(k_start(k), 128), _SLAB)],
            buf, sem)

    def wait_slab(buf, sem):
        pltpu.make_async_copy(
            xt_hbm.at[:, pl.ds(0, _SLAB)], buf, sem).wait()

    def row_out(p):
        pd = rowpend_s[0]
        pltpu.async_copy(
            rowbuf.at[pl.ds(jnp.where(pd == _RING, 0, pd), 1)],
            raw_hbm.at[pl.ds(p, 1)], sem_row)
        rowpend_s[0] = jnp.where(pd == _RING, 1, pd + 1)

    def ring_prepare():
        pd = rowpend_s[0]

        @pl.when(pd == _RING)
        def _absorb():
            pltpu.make_async_copy(
                rowbuf, raw_hbm.at[pl.ds(0, _RING)], sem_row).wait()

        return jnp.where(pd == _RING, 0, pd)

    def process_region(k, gather_from_tail, buf):
        def scan_vreg(v, carry):
            base = lax.shift_left(k, 8) + v * _LANES
            sv = slot[pl.ds(base, _LANES)]
            hits = sv >= 0
            cnt = plsc.all_reduce_population_count(hits)[0]

            @pl.when(cnt > 0)
            def _hits():
                mvec = _iota16() + v * _LANES
                plsc.store_compressed(
                    hit_m.at[pl.ds(0, _LANES)], mvec, mask=hits)
                plsc.store_compressed(
                    hit_p.at[pl.ds(0, _LANES)], sv, mask=hits)

                def do_hit(h, carry):
                    m = hit_m[pl.ds(h, _LANES)][0]
                    p = hit_p[pl.ds(h, _LANES)][0]
                    slot_r = ring_prepare()
                    if gather_from_tail:
                        for j in range(_TOKEN_DIM // _LANES):
                            sl = pl.ds(j * _LANES, _LANES)
                            rowbuf[slot_r, sl] = tail_v[m, sl]
                    else:
                        col = jnp.full((_LANES,), m, dtype=jnp.int32)
                        for j in range(_TOKEN_DIM // _LANES):
                            vals = plsc.load_gather(
                                buf, [_iota16() + j * _LANES, col])
                            rowbuf[slot_r, pl.ds(j * _LANES, _LANES)] = vals
                    row_out(p)
                    return carry

                lax.fori_loop(0, cnt, do_hit, 0)

            return carry

        lax.fori_loop(0, _SLAB // _LANES, scan_vreg, 0)

    bufs = (slab_a, slab_b, slab_c, slab_d)
    sems = (sem_a, sem_b, sem_c, sem_d)
    nquads = lax.shift_right_logical(my_nfull + 3, 2)

    for q in range(4):
        issue_k(q, bufs[q], sems[q])

    def quadloop(i, carry):
        k0 = 4 * i
        for q in range(4):
            wait_slab(bufs[q], sems[q])

            @pl.when(k0 + q < my_nfull)
            def _p(q=q):
                process_region(k0 + q, False, bufs[q])

            @pl.when(i + 1 < nquads)
            def _pre(q=q):
                issue_k(k0 + 4 + q, bufs[q], sems[q])

        return carry

    lax.fori_loop(0, nquads, quadloop, 0)

    @pl.when(wid == _TAIL_OWNER)
    def _tail():
        process_region((_NSLAB - 1) // _NW, True, slab_a)

    def drain(r, carry):
        @pl.when(r < rowpend_s[0])
        def _w():
            pltpu.make_async_copy(
                rowbuf.at[pl.ds(0, 1)], raw_hbm.at[pl.ds(0, 1)], sem_row
            ).wait()
        return carry

    lax.fori_loop(0, _RING, drain, 0)

    def dup_copy(i, carry):
        src = pend_src[pl.ds(i, _LANES)][0]
        dst = pend_dst[pl.ds(i, _LANES)][0]
        pltpu.async_copy(
            raw_hbm.at[pl.ds(src, 1)], raw_hbm.at[pl.ds(dst, 1)], sem_p)
        pltpu.make_async_copy(
            raw_hbm.at[pl.ds(0, 1)], raw_hbm.at[pl.ds(0, 1)], sem_p).wait()
        return carry

    def dup_chunk(sp, carry):
        pltpu.sync_copy(pend_hbm.at[0, pl.ds(sp * _PCAP, _PCAP)],
                        pend_src.at[pl.ds(0, _PCAP)])
        pltpu.sync_copy(pend_hbm.at[1, pl.ds(sp * _PCAP, _PCAP)],
                        pend_dst.at[pl.ds(0, _PCAP)])
        lax.fori_loop(0, _PCAP, dup_copy, 0)
        return carry

    lax.fori_loop(0, nspill, dup_chunk, 0)
    lax.fori_loop(0, pcur, dup_copy, 0)


def _add_body(raw_hbm, pos_hbm, out_hbm, rows_v, pos_v):
    c = lax.axis_index("c")
    s = lax.axis_index("s")
    wid = s * _NC + c
    per_w = _TOK // _NW
    base = wid * per_w
    pos_base = (wid % (_N_CTX // per_w)) * per_w
    q = per_w // 4

    for h in range(4):
        pltpu.sync_copy(raw_hbm.at[pl.ds(base + h * q, q)], rows_v)
        pltpu.sync_copy(pos_hbm.at[pl.ds(pos_base + h * q, q)], pos_v)

        def add_row(r, carry):
            for j in range(_TOKEN_DIM // _LANES):
                sl = pl.ds(j * _LANES, _LANES)
                rows_v[r, sl] = rows_v[r, sl] + pos_v[r, sl]
            return carry

        lax.fori_loop(0, q, add_row, 0)
        pltpu.sync_copy(rows_v, out_hbm.at[pl.ds(base + h * q, q)])


@jax.jit
def _conditioner(tokens_flat, xt, tail, pos_emb):
    mesh = plsc.VectorSubcoreMesh(core_axis_name="c", subcore_axis_name="s")
    raw, _ = functools.partial(
        pl.kernel,
        mesh=mesh,
        compiler_params=pltpu.CompilerParams(needs_layout_passes=False),
        out_type=(
            jax.ShapeDtypeStruct((_TOK, _TOKEN_DIM), jnp.float32),
            jax.ShapeDtypeStruct((2, _TOK), jnp.int32),
        ),
        scratch_types=[
            pltpu.VMEM((_NSLOT + _LANES,), jnp.int32),
            pltpu.VMEM((_CHUNK,), jnp.int32),
            pltpu.VMEM((_TOKEN_DIM, _SLAB), jnp.float32),
            pltpu.VMEM((_TOKEN_DIM, _SLAB), jnp.float32),
            pltpu.VMEM((_TOKEN_DIM, _SLAB), jnp.float32),
            pltpu.VMEM((_TOKEN_DIM, _SLAB), jnp.float32),
            pltpu.VMEM((2 * _LANES,), jnp.int32),
            pltpu.VMEM((2 * _LANES,), jnp.int32),
            pltpu.VMEM((_RING, _TOKEN_DIM), jnp.float32),
            pltpu.VMEM((64, _TOKEN_DIM), jnp.float32),
            pltpu.VMEM((_PCAP + _LANES,), jnp.int32),
            pltpu.VMEM((_PCAP + _LANES,), jnp.int32),
            pltpu.SMEM((1,), jnp.int32),
            pltpu.SemaphoreType.DMA,
            pltpu.SemaphoreType.DMA,
            pltpu.SemaphoreType.DMA,
            pltpu.SemaphoreType.DMA,
            pltpu.SemaphoreType.DMA,
            pltpu.SemaphoreType.DMA,
        ],
    )(_gather_body)(tokens_flat, xt, tail)

    out = functools.partial(
        pl.kernel,
        mesh=mesh,
        compiler_params=pltpu.CompilerParams(needs_layout_passes=False),
        out_type=jax.ShapeDtypeStruct((_TOK, _TOKEN_DIM), jnp.float32),
        scratch_types=[
            pltpu.VMEM((_TOK // _NW // 4, _TOKEN_DIM), jnp.float32),
            pltpu.VMEM((_TOK // _NW // 4, _TOKEN_DIM), jnp.float32),
        ],
    )(_add_body)(raw, pos_emb)
    return out


def kernel(tokens, x_emb, pos_emb):
    tokens_flat = tokens.reshape(-1).astype(jnp.int32)
    tail = x_emb[_TAIL_BASE:, :]
    out = _conditioner(tokens_flat, x_emb.T, tail, pos_emb)
    return out.reshape(_B, _N_CTX, _TOKEN_DIM)
